# Initial kernel scaffold; baseline (speedup 1.0000x reference)
import functools

import jax
import jax.numpy as jnp
from jax import lax
from jax.experimental import pallas as pl
from jax.experimental.pallas import tpu as pltpu
from jax.experimental.pallas import tpu_sc as plsc

N = 10000
E = 320000
H = 128
G = 16
L = 8
EPS = 1e-5

NP = 10240
NB = 512
NBLK = NP // NB
NC, NS = 2, 16
NW = NC * NS
CHUNK = 128
NCH = 80
EPW = CHUNK * NCH
EPAD = EPW * NW
RPT = NP // NS

_mesh = plsc.VectorSubcoreMesh(
    core_axis_name="c", subcore_axis_name="s", num_cores=NC, num_subcores=NS)


@functools.partial(
    pl.kernel,
    out_type=jax.ShapeDtypeStruct((NC, NP, H), jnp.float32),
    mesh=_mesh,
    scratch_types=[
        pltpu.VMEM((CHUNK, H), jnp.float32),
        pltpu.VMEM((CHUNK, H), jnp.float32),
        pltpu.VMEM((NCH, CHUNK), jnp.int32),
        pltpu.VMEM((NCH, CHUNK), jnp.int32),
        pltpu.VMEM_SHARED((NP, H), jnp.float32),
        pltpu.SemaphoreType.DMA,
        pltpu.SemaphoreType.DMA,
    ],
)
def _edge_agg(rows_hbm, src_hbm, dst_hbm, zeros_hbm, out_hbm,
              buf_a, buf_b, sidx, didx, acc, sem_a, sem_b):
    c = lax.axis_index("c")
    s = lax.axis_index("s")
    w = c * NS + s
    r0 = s * RPT
    pltpu.sync_copy(zeros_hbm.at[pl.ds(r0, RPT)], acc.at[pl.ds(r0, RPT)])
    pltpu.sync_copy(src_hbm.at[w], sidx)
    pltpu.sync_copy(dst_hbm.at[w], didx)
    plsc.subcore_barrier()

    pltpu.async_copy(rows_hbm.at[sidx.at[0]], buf_a, sem_a)

    def _pair(t, carry):
        j = 2 * t
        pltpu.async_copy(rows_hbm.at[sidx.at[j + 1]], buf_b, sem_b)
        pltpu.make_async_copy(rows_hbm.at[sidx.at[0]], buf_a, sem_a).wait()
        pltpu.sync_copy(buf_a, acc.at[didx.at[j]], add=True)

        @pl.when(j + 2 < NCH)
        def _():
            pltpu.async_copy(rows_hbm.at[sidx.at[j + 2]], buf_a, sem_a)

        pltpu.make_async_copy(rows_hbm.at[sidx.at[0]], buf_b, sem_b).wait()
        pltpu.sync_copy(buf_b, acc.at[didx.at[j + 1]], add=True)
        return carry

    lax.fori_loop(0, NCH // 2, _pair, 0)
    plsc.subcore_barrier()
    pltpu.sync_copy(acc.at[pl.ds(r0, RPT)], out_hbm.at[c, pl.ds(r0, RPT)])


def _mm(a, w):
    return lax.dot_general(a, w, (((1,), (0,)), ((), ())),
                           preferred_element_type=jnp.float32,
                           precision=lax.Precision.HIGHEST)


def _row_mask(i):
    return (lax.broadcasted_iota(jnp.int32, (NB, 1), 0) + i * NB) < N


def _init_body(x_ref, cnt_ref, wi_ref, bi_ref, w0_ref,
               dinv_ref, h_ref, hws_ref):
    i = pl.program_id(0)
    cnt = cnt_ref[0, :, 0:1] + cnt_ref[1, :, 0:1]
    dinv = jnp.where(_row_mask(i), lax.rsqrt(cnt + 1.0), 0.0)
    h = _mm(x_ref[...], wi_ref[...]) + bi_ref[...]
    dinv_ref[...] = dinv
    h_ref[...] = h
    hws_ref[...] = _mm(h, w0_ref[...]) * dinv


def _init_call(xp, counts, Wi, bi, W0):
    return pl.pallas_call(
        _init_body,
        grid=(NBLK,),
        in_specs=[
            pl.BlockSpec((NB, H), lambda i: (i, 0)),
            pl.BlockSpec((NC, NB, H), lambda i: (0, i, 0)),
            pl.BlockSpec((H, H), lambda i: (0, 0)),
            pl.BlockSpec((1, H), lambda i: (0, 0)),
            pl.BlockSpec((H, H), lambda i: (0, 0)),
        ],
        out_specs=[
            pl.BlockSpec((NB, 1), lambda i: (i, 0)),
            pl.BlockSpec((NB, H), lambda i: (i, 0)),
            pl.BlockSpec((NB, H), lambda i: (i, 0)),
        ],
        out_shape=[
            jax.ShapeDtypeStruct((NP, 1), jnp.float32),
            jax.ShapeDtypeStruct((NP, H), jnp.float32),
            jax.ShapeDtypeStruct((NP, H), jnp.float32),
        ],
    )(xp, counts, Wi, bi, W0)


def _stats_body(acc_ref, hws_ref, dinv_ref, b_ref, agg_ref, s_ref, q_ref):
    i = pl.program_id(0)
    a = (acc_ref[0] + acc_ref[1] + hws_ref[...]) * dinv_ref[...] + b_ref[...]
    a = jnp.where(_row_mask(i), a, 0.0)
    agg_ref[...] = a

    @pl.when(i == 0)
    def _():
        s_ref[...] = jnp.zeros_like(s_ref)
        q_ref[...] = jnp.zeros_like(q_ref)

    s_ref[...] += jnp.sum(a, axis=0, keepdims=True)
    q_ref[...] += jnp.sum(a * a, axis=0, keepdims=True)


def _stats_call(acc, hws, dinv, b):
    return pl.pallas_call(
        _stats_body,
        grid=(NBLK,),
        in_specs=[
            pl.BlockSpec((NC, NB, H), lambda i: (0, i, 0)),
            pl.BlockSpec((NB, H), lambda i: (i, 0)),
            pl.BlockSpec((NB, 1), lambda i: (i, 0)),
            pl.BlockSpec((1, H), lambda i: (0, 0)),
        ],
        out_specs=[
            pl.BlockSpec((NB, H), lambda i: (i, 0)),
            pl.BlockSpec((1, H), lambda i: (0, 0)),
            pl.BlockSpec((1, H), lambda i: (0, 0)),
        ],
        out_shape=[
            jax.ShapeDtypeStruct((NP, H), jnp.float32),
            jax.ShapeDtypeStruct((1, H), jnp.float32),
            jax.ShapeDtypeStruct((1, H), jnp.float32),
        ],
    )(acc, hws, dinv, b)


def _bn(agg_ref, s_ref, q_ref, g_ref, be_ref):
    mu = s_ref[...] * (1.0 / N)
    var = q_ref[...] * (1.0 / N) - mu * mu
    rstd = lax.rsqrt(var + EPS)
    return (agg_ref[...] - mu) * rstd * g_ref[...] + be_ref[...]


def _fuse1_body(agg_ref, s_ref, q_ref, g_ref, be_ref, w_ref, dinv_ref,
                hws_ref):
    y = jnp.maximum(_bn(agg_ref, s_ref, q_ref, g_ref, be_ref), 0.0)
    hws_ref[...] = _mm(y, w_ref[...]) * dinv_ref[...]


def _fuse1_call(agg, s, q, g, be, W, dinv):
    return pl.pallas_call(
        _fuse1_body,
        grid=(NBLK,),
        in_specs=[
            pl.BlockSpec((NB, H), lambda i: (i, 0)),
            pl.BlockSpec((1, H), lambda i: (0, 0)),
            pl.BlockSpec((1, H), lambda i: (0, 0)),
            pl.BlockSpec((1, H), lambda i: (0, 0)),
            pl.BlockSpec((1, H), lambda i: (0, 0)),
            pl.BlockSpec((H, H), lambda i: (0, 0)),
            pl.BlockSpec((NB, 1), lambda i: (i, 0)),
        ],
        out_specs=pl.BlockSpec((NB, H), lambda i: (i, 0)),
        out_shape=jax.ShapeDtypeStruct((NP, H), jnp.float32),
    )(agg, s, q, g, be, W, dinv)


def _fuse2_body(agg_ref, s_ref, q_ref, g_ref, be_ref, hprev_ref, w_ref,
                dinv_ref, h_ref, hws_ref):
    y = _bn(agg_ref, s_ref, q_ref, g_ref, be_ref) + hprev_ref[...]
    h_ref[...] = y
    hws_ref[...] = _mm(y, w_ref[...]) * dinv_ref[...]


def _fuse2_call(agg, s, q, g, be, hprev, W, dinv):
    return pl.pallas_call(
        _fuse2_body,
        grid=(NBLK,),
        in_specs=[
            pl.BlockSpec((NB, H), lambda i: (i, 0)),
            pl.BlockSpec((1, H), lambda i: (0, 0)),
            pl.BlockSpec((1, H), lambda i: (0, 0)),
            pl.BlockSpec((1, H), lambda i: (0, 0)),
            pl.BlockSpec((1, H), lambda i: (0, 0)),
            pl.BlockSpec((NB, H), lambda i: (i, 0)),
            pl.BlockSpec((H, H), lambda i: (0, 0)),
            pl.BlockSpec((NB, 1), lambda i: (i, 0)),
        ],
        out_specs=[
            pl.BlockSpec((NB, H), lambda i: (i, 0)),
            pl.BlockSpec((NB, H), lambda i: (i, 0)),
        ],
        out_shape=[
            jax.ShapeDtypeStruct((NP, H), jnp.float32),
            jax.ShapeDtypeStruct((NP, H), jnp.float32),
        ],
    )(agg, s, q, g, be, hprev, W, dinv)


def _pool_body(h_ref, b_ref, wf1_ref, bf1_ref, wf2_ref, bf2_ref, wf3_ref,
               bf3_ref, out_ref):
    hm = h_ref[...]
    bidx = b_ref[...]
    seg = lax.broadcasted_iota(jnp.int32, (NP, G), 1)
    oh = (bidx == seg).astype(jnp.float32)
    ssum = lax.dot_general(oh, hm, (((0,), (0,)), ((), ())),
                           preferred_element_type=jnp.float32,
                           precision=lax.Precision.HIGHEST)
    counts = jnp.sum(oh, axis=0)[:, None]
    smean = ssum / jnp.maximum(counts, 1.0)
    neg = jnp.float32(-jnp.inf)
    parts = [jnp.max(jnp.where(bidx == g, hm, neg), axis=0, keepdims=True)
             for g in range(G)]
    smax = jnp.concatenate(parts, axis=0)
    z = jnp.concatenate([smean, smax, ssum], axis=1)
    z = jnp.maximum(_mm(z, wf1_ref[...]) + bf1_ref[...], 0.0)
    z = jnp.maximum(_mm(z, wf2_ref[...]) + bf2_ref[...], 0.0)
    out_ref[...] = _mm(z, wf3_ref[...]) + bf3_ref[...]


def _pool_call(h, bp, Wf1, bf1, Wf2, bf2, Wf3, bf3):
    return pl.pallas_call(
        _pool_body,
        out_shape=jax.ShapeDtypeStruct((G, 1), jnp.float32),
    )(h, bp, Wf1, bf1, Wf2, bf2, Wf3, bf3)


def kernel(x, edge_index, batch, Wi, bi, Wc1, bc1, g1, be1,
           Wc2, bc2, g2, be2, Wf1, bf1, Wf2, bf2, Wf3, bf3):
    pad = EPAD - E
    padv = jnp.full((pad,), N, jnp.int32)
    srcw = jnp.concatenate([edge_index[0], padv]).reshape(NW, NCH, CHUNK)
    dstw = jnp.concatenate([edge_index[1], padv]).reshape(NW, NCH, CHUNK)
    zeros = jnp.zeros((NP, H), jnp.float32)
    ones = jnp.ones((NP, H), jnp.float32)

    counts = _edge_agg(ones, srcw, dstw, zeros)
    xp = jnp.pad(x, ((0, NP - N), (0, 0)))
    dinv, h, hws = _init_call(xp, counts, Wi, bi.reshape(1, H), Wc1[0])

    for i in range(L):
        acc = _edge_agg(hws, srcw, dstw, zeros)
        agg, s, q = _stats_call(acc, hws, dinv, bc1[i].reshape(1, H))
        hws = _fuse1_call(agg, s, q, g1[i].reshape(1, H),
                          be1[i].reshape(1, H), Wc2[i], dinv)
        acc = _edge_agg(hws, srcw, dstw, zeros)
        agg, s, q = _stats_call(acc, hws, dinv, bc2[i].reshape(1, H))
        w_next = Wc1[i + 1] if i + 1 < L else Wc1[0]
        h, hws = _fuse2_call(agg, s, q, g2[i].reshape(1, H),
                             be2[i].reshape(1, H), h, w_next, dinv)

    bp = jnp.concatenate([batch, jnp.full((NP - N,), G, jnp.int32)])
    return _pool_call(h, bp.reshape(NP, 1), Wf1, bf1.reshape(1, 2 * H),
                      Wf2, bf2.reshape(1, H), Wf3, bf3.reshape(1, 1))


# speedup vs baseline: 5.3725x; 5.3725x over previous
import functools

import jax
import jax.numpy as jnp
from jax import lax
from jax.experimental import pallas as pl
from jax.experimental.pallas import tpu as pltpu
from jax.experimental.pallas import tpu_sc as plsc

N = 10000
E = 320000
H = 128
G = 16
L = 8
EPS = 1e-5

NP = 10240
NB = 512
NBLK = NP // NB
NC, NS = 2, 16
NW = NC * NS
CHUNK = 128
NCH = 80
EPW = CHUNK * NCH
EPAD = EPW * NW
RPT = NP // NS

@functools.cache
def _edge_agg_build():
    mesh = plsc.VectorSubcoreMesh(
        core_axis_name="c", subcore_axis_name="s",
        num_cores=NC, num_subcores=NS)
    return functools.partial(
        pl.kernel,
        out_type=jax.ShapeDtypeStruct((NC, NP, H), jnp.float32),
        mesh=mesh,
        scratch_types=[
            pltpu.VMEM((CHUNK, H), jnp.float32),
            pltpu.VMEM((CHUNK, H), jnp.float32),
            pltpu.VMEM((NCH // 2, CHUNK), jnp.int32),
            pltpu.VMEM((NCH // 2, CHUNK), jnp.int32),
            pltpu.VMEM_SHARED((NP, H), jnp.float32),
            pltpu.SemaphoreType.DMA,
            pltpu.SemaphoreType.DMA,
        ],
    )(_edge_agg_body)


def _edge_agg(rows, srcw, dstw, zeros):
    return _edge_agg_build()(rows, srcw, dstw, zeros)


def _edge_agg_body(rows_hbm, src_hbm, dst_hbm, zeros_hbm, out_hbm,
                   buf_a, buf_b, sidx, didx, acc, sem_a, sem_b):
    c = lax.axis_index("c")
    s = lax.axis_index("s")
    w = c * NS + s
    r0 = s * RPT
    pltpu.sync_copy(zeros_hbm.at[pl.ds(r0, RPT)], acc.at[pl.ds(r0, RPT)])
    plsc.subcore_barrier()

    hc = NCH // 2
    for p in range(2):
        pltpu.sync_copy(src_hbm.at[w, pl.ds(p * hc, hc)], sidx)
        pltpu.sync_copy(dst_hbm.at[w, pl.ds(p * hc, hc)], didx)
        pltpu.async_copy(rows_hbm.at[sidx.at[0]], buf_a, sem_a)

        def _pair(t, carry):
            j = 2 * t
            pltpu.async_copy(rows_hbm.at[sidx.at[j + 1]], buf_b, sem_b)
            pltpu.make_async_copy(rows_hbm.at[sidx.at[j]], buf_a, sem_a).wait()
            pltpu.sync_copy(buf_a, acc.at[didx.at[j]], add=True)
            pltpu.async_copy(rows_hbm.at[sidx.at[j + 2]], buf_a, sem_a)
            pltpu.make_async_copy(rows_hbm.at[sidx.at[j + 1]], buf_b, sem_b).wait()
            pltpu.sync_copy(buf_b, acc.at[didx.at[j + 1]], add=True)
            return carry

        lax.fori_loop(0, hc // 2 - 1, _pair, 0)
        jl = hc - 2
        pltpu.async_copy(rows_hbm.at[sidx.at[jl + 1]], buf_b, sem_b)
        pltpu.make_async_copy(rows_hbm.at[sidx.at[jl]], buf_a, sem_a).wait()
        pltpu.sync_copy(buf_a, acc.at[didx.at[jl]], add=True)
        pltpu.make_async_copy(rows_hbm.at[sidx.at[jl + 1]], buf_b, sem_b).wait()
        pltpu.sync_copy(buf_b, acc.at[didx.at[jl + 1]], add=True)
    plsc.subcore_barrier()
    pltpu.sync_copy(acc.at[pl.ds(r0, RPT)], out_hbm.at[c, pl.ds(r0, RPT)])


def _mm(a, w):
    return lax.dot_general(a, w, (((1,), (0,)), ((), ())),
                           preferred_element_type=jnp.float32)


def _row_mask(i):
    return (lax.broadcasted_iota(jnp.int32, (NB, 1), 0) + i * NB) < N


def _init_body(x_ref, cnt_ref, wi_ref, bi_ref, w0_ref,
               dinv_ref, h_ref, hws_ref):
    i = pl.program_id(0)
    cnt = cnt_ref[0, :, 0:1] + cnt_ref[1, :, 0:1]
    dinv = jnp.where(_row_mask(i), lax.rsqrt(cnt + 1.0), 0.0)
    h = _mm(x_ref[...], wi_ref[...]) + bi_ref[...]
    dinv_ref[...] = dinv
    h_ref[...] = h
    hws_ref[...] = _mm(h, w0_ref[...]) * dinv


def _init_call(xp, counts, Wi, bi, W0):
    return pl.pallas_call(
        _init_body,
        grid=(NBLK,),
        in_specs=[
            pl.BlockSpec((NB, H), lambda i: (i, 0)),
            pl.BlockSpec((NC, NB, H), lambda i: (0, i, 0)),
            pl.BlockSpec((H, H), lambda i: (0, 0)),
            pl.BlockSpec((1, H), lambda i: (0, 0)),
            pl.BlockSpec((H, H), lambda i: (0, 0)),
        ],
        out_specs=[
            pl.BlockSpec((NB, 1), lambda i: (i, 0)),
            pl.BlockSpec((NB, H), lambda i: (i, 0)),
            pl.BlockSpec((NB, H), lambda i: (i, 0)),
        ],
        out_shape=[
            jax.ShapeDtypeStruct((NP, 1), jnp.float32),
            jax.ShapeDtypeStruct((NP, H), jnp.float32),
            jax.ShapeDtypeStruct((NP, H), jnp.float32),
        ],
    )(xp, counts, Wi, bi, W0)


def _stats_body(acc_ref, hws_ref, dinv_ref, b_ref, agg_ref, s_ref):
    i = pl.program_id(0)
    a = (acc_ref[0] + acc_ref[1] + hws_ref[...]) * dinv_ref[...] + b_ref[...]
    a = jnp.where(_row_mask(i), a, 0.0)
    agg_ref[...] = a

    @pl.when(i == 0)
    def _():
        s_ref[...] = jnp.zeros_like(s_ref)

    s_ref[...] += jnp.sum(a, axis=0, keepdims=True)


def _stats_call(acc, hws, dinv, b):
    return pl.pallas_call(
        _stats_body,
        grid=(NBLK,),
        in_specs=[
            pl.BlockSpec((NC, NB, H), lambda i: (0, i, 0)),
            pl.BlockSpec((NB, H), lambda i: (i, 0)),
            pl.BlockSpec((NB, 1), lambda i: (i, 0)),
            pl.BlockSpec((1, H), lambda i: (0, 0)),
        ],
        out_specs=[
            pl.BlockSpec((NB, H), lambda i: (i, 0)),
            pl.BlockSpec((1, H), lambda i: (0, 0)),
        ],
        out_shape=[
            jax.ShapeDtypeStruct((NP, H), jnp.float32),
            jax.ShapeDtypeStruct((1, H), jnp.float32),
        ],
    )(acc, hws, dinv, b)


def _var_body(agg_ref, s_ref, q_ref):
    i = pl.program_id(0)
    d = jnp.where(_row_mask(i), agg_ref[...] - s_ref[...] * (1.0 / N), 0.0)

    @pl.when(i == 0)
    def _():
        q_ref[...] = jnp.zeros_like(q_ref)

    q_ref[...] += jnp.sum(d * d, axis=0, keepdims=True)


def _var_call(agg, s):
    return pl.pallas_call(
        _var_body,
        grid=(NBLK,),
        in_specs=[
            pl.BlockSpec((NB, H), lambda i: (i, 0)),
            pl.BlockSpec((1, H), lambda i: (0, 0)),
        ],
        out_specs=pl.BlockSpec((1, H), lambda i: (0, 0)),
        out_shape=jax.ShapeDtypeStruct((1, H), jnp.float32),
    )(agg, s)


def _bn(agg_ref, s_ref, q_ref, g_ref, be_ref):
    mu = s_ref[...] * (1.0 / N)
    var = q_ref[...] * (1.0 / N)
    return ((agg_ref[...] - mu) / jnp.sqrt(var + EPS)) * g_ref[...]         + be_ref[...]


def _fuse1_body(agg_ref, s_ref, q_ref, g_ref, be_ref, w_ref, dinv_ref,
                hws_ref):
    y = jnp.maximum(_bn(agg_ref, s_ref, q_ref, g_ref, be_ref), 0.0)
    hws_ref[...] = _mm(y, w_ref[...]) * dinv_ref[...]


def _fuse1_call(agg, s, q, g, be, W, dinv):
    return pl.pallas_call(
        _fuse1_body,
        grid=(NBLK,),
        in_specs=[
            pl.BlockSpec((NB, H), lambda i: (i, 0)),
            pl.BlockSpec((1, H), lambda i: (0, 0)),
            pl.BlockSpec((1, H), lambda i: (0, 0)),
            pl.BlockSpec((1, H), lambda i: (0, 0)),
            pl.BlockSpec((1, H), lambda i: (0, 0)),
            pl.BlockSpec((H, H), lambda i: (0, 0)),
            pl.BlockSpec((NB, 1), lambda i: (i, 0)),
        ],
        out_specs=pl.BlockSpec((NB, H), lambda i: (i, 0)),
        out_shape=jax.ShapeDtypeStruct((NP, H), jnp.float32),
    )(agg, s, q, g, be, W, dinv)


def _fuse2_body(agg_ref, s_ref, q_ref, g_ref, be_ref, hprev_ref, w_ref,
                dinv_ref, h_ref, hws_ref):
    y = _bn(agg_ref, s_ref, q_ref, g_ref, be_ref) + hprev_ref[...]
    h_ref[...] = y
    hws_ref[...] = _mm(y, w_ref[...]) * dinv_ref[...]


def _fuse2_call(agg, s, q, g, be, hprev, W, dinv):
    return pl.pallas_call(
        _fuse2_body,
        grid=(NBLK,),
        in_specs=[
            pl.BlockSpec((NB, H), lambda i: (i, 0)),
            pl.BlockSpec((1, H), lambda i: (0, 0)),
            pl.BlockSpec((1, H), lambda i: (0, 0)),
            pl.BlockSpec((1, H), lambda i: (0, 0)),
            pl.BlockSpec((1, H), lambda i: (0, 0)),
            pl.BlockSpec((NB, H), lambda i: (i, 0)),
            pl.BlockSpec((H, H), lambda i: (0, 0)),
            pl.BlockSpec((NB, 1), lambda i: (i, 0)),
        ],
        out_specs=[
            pl.BlockSpec((NB, H), lambda i: (i, 0)),
            pl.BlockSpec((NB, H), lambda i: (i, 0)),
        ],
        out_shape=[
            jax.ShapeDtypeStruct((NP, H), jnp.float32),
            jax.ShapeDtypeStruct((NP, H), jnp.float32),
        ],
    )(agg, s, q, g, be, hprev, W, dinv)


def _pool_body(h_ref, b_ref, wf1_ref, bf1_ref, wf2_ref, bf2_ref, wf3_ref,
               bf3_ref, out_ref):
    hm = h_ref[...]
    bidx = b_ref[...]
    seg = lax.broadcasted_iota(jnp.int32, (NP, G), 1)
    oh = (bidx == seg).astype(jnp.float32)
    ssum = lax.dot_general(oh, hm, (((0,), (0,)), ((), ())),
                           preferred_element_type=jnp.float32,
                           precision=lax.Precision.HIGHEST)
    counts = jnp.sum(oh, axis=0)[:, None]
    smean = ssum / jnp.maximum(counts, 1.0)
    neg = jnp.float32(-jnp.inf)
    parts = [jnp.max(jnp.where(bidx == g, hm, neg), axis=0, keepdims=True)
             for g in range(G)]
    smax = jnp.concatenate(parts, axis=0)
    z = jnp.concatenate([smean, smax, ssum], axis=1)
    z = jnp.maximum(_mm(z, wf1_ref[...]) + bf1_ref[...], 0.0)
    z = jnp.maximum(_mm(z, wf2_ref[...]) + bf2_ref[...], 0.0)
    out_ref[...] = _mm(z, wf3_ref[...]) + bf3_ref[...]


def _pool_call(h, bp, Wf1, bf1, Wf2, bf2, Wf3, bf3):
    return pl.pallas_call(
        _pool_body,
        out_shape=jax.ShapeDtypeStruct((G, 1), jnp.float32),
    )(h, bp, Wf1, bf1, Wf2, bf2, Wf3, bf3)


def kernel(x, edge_index, batch, Wi, bi, Wc1, bc1, g1, be1,
           Wc2, bc2, g2, be2, Wf1, bf1, Wf2, bf2, Wf3, bf3):
    pad = EPAD - E
    padv = jnp.full((pad,), N, jnp.int32)
    srcw = jnp.concatenate([edge_index[0], padv]).reshape(NW, NCH, CHUNK)
    dstw = jnp.concatenate([edge_index[1], padv]).reshape(NW, NCH, CHUNK)
    zeros = jnp.zeros((NP, H), jnp.float32)
    ones = jnp.ones((NP, H), jnp.float32)

    counts = _edge_agg(ones, srcw, dstw, zeros)
    xp = jnp.pad(x, ((0, NP - N), (0, 0)))
    dinv, h, hws = _init_call(xp, counts, Wi, bi.reshape(1, H), Wc1[0])

    for i in range(L):
        acc = _edge_agg(hws, srcw, dstw, zeros)
        agg, s = _stats_call(acc, hws, dinv, bc1[i].reshape(1, H))
        q = _var_call(agg, s)
        hws = _fuse1_call(agg, s, q, g1[i].reshape(1, H),
                          be1[i].reshape(1, H), Wc2[i], dinv)
        acc = _edge_agg(hws, srcw, dstw, zeros)
        agg, s = _stats_call(acc, hws, dinv, bc2[i].reshape(1, H))
        q = _var_call(agg, s)
        w_next = Wc1[i + 1] if i + 1 < L else Wc1[0]
        h, hws = _fuse2_call(agg, s, q, g2[i].reshape(1, H),
                             be2[i].reshape(1, H), h, w_next, dinv)

    bp = jnp.concatenate([batch, jnp.full((NP - N,), G, jnp.int32)])
    return _pool_call(h, bp.reshape(NP, 1), Wf1, bf1.reshape(1, 2 * H),
                      Wf2, bf2.reshape(1, H), Wf3, bf3.reshape(1, 1))
